# Initial kernel scaffold; baseline (speedup 1.0000x reference)
import functools

import jax
import jax.numpy as jnp
from jax import lax
from jax.experimental import pallas as pl
from jax.experimental.pallas import tpu as pltpu
from jax.experimental.pallas import tpu_sc as plsc

N_POINTS = 320000
D = 128
V = 10000
NC = 2
NS = 16
NW = NC * NS
ROWS_PER_TILE = N_POINTS // NW
CHUNK = 125
CHUNKS_PER_TILE = ROWS_PER_TILE // CHUNK
V_PER_TILE = V // NS
CW = 16


def _sc_partial_sums(features, index2d, zero_sums, zero_cnts, ones_blk):
    mesh = plsc.VectorSubcoreMesh(core_axis_name="c", subcore_axis_name="s")

    @functools.partial(
        pl.kernel,
        out_type=(
            jax.ShapeDtypeStruct((NC, V, D), jnp.float32),
            jax.ShapeDtypeStruct((NC, V, CW), jnp.float32),
        ),
        mesh=mesh,
        scratch_types=(
            pltpu.VMEM((CHUNK, D), jnp.float32),
            pltpu.VMEM((CHUNK,), jnp.int32),
            pltpu.VMEM((CHUNK, CW), jnp.float32),
            pltpu.VMEM_SHARED((V, D), jnp.float32),
            pltpu.VMEM_SHARED((V, CW), jnp.float32),
        ),
    )
    def body(feat_hbm, idx_hbm, zs_hbm, zc_hbm, ones_hbm, sums_out, cnts_out,
             rows_v, idx_v, ones_v, acc_s, cacc_s):
        c = lax.axis_index("c")
        s = lax.axis_index("s")
        wid = s * NC + c

        pltpu.sync_copy(zs_hbm, acc_s.at[pl.ds(s * V_PER_TILE, V_PER_TILE)])
        pltpu.sync_copy(zc_hbm, cacc_s.at[pl.ds(s * V_PER_TILE, V_PER_TILE)])
        pltpu.sync_copy(ones_hbm, ones_v)
        plsc.subcore_barrier()

        def chunk_body(k, carry):
            row0 = wid * ROWS_PER_TILE + k * CHUNK
            g = wid * CHUNKS_PER_TILE + k
            pltpu.sync_copy(feat_hbm.at[pl.ds(row0, CHUNK)], rows_v)
            pltpu.sync_copy(idx_hbm.at[g], idx_v)
            pltpu.sync_copy(rows_v, acc_s.at[idx_v], add=True)
            pltpu.sync_copy(ones_v, cacc_s.at[idx_v], add=True)
            return carry

        lax.fori_loop(0, CHUNKS_PER_TILE, chunk_body, 0)
        plsc.subcore_barrier()

        v0 = s * V_PER_TILE
        pltpu.sync_copy(acc_s.at[pl.ds(v0, V_PER_TILE)],
                        sums_out.at[c, pl.ds(v0, V_PER_TILE)])
        pltpu.sync_copy(cacc_s.at[pl.ds(v0, V_PER_TILE)],
                        cnts_out.at[c, pl.ds(v0, V_PER_TILE)])

    return body(features, index2d, zero_sums, zero_cnts, ones_blk)


def _combine_body(sums_ref, cnts_ref, out_ref):
    total = sums_ref[0] + sums_ref[1]
    cnt = cnts_ref[0, :, 0] + cnts_ref[1, :, 0]
    out_ref[...] = total / jnp.clip(cnt, 1.0, None)[:, None]


def kernel(features, index):
    index2d = index.astype(jnp.int32).reshape(N_POINTS // CHUNK, CHUNK)
    zero_sums = jnp.zeros((V_PER_TILE, D), jnp.float32)
    zero_cnts = jnp.zeros((V_PER_TILE, CW), jnp.float32)
    ones_blk = jnp.ones((CHUNK, CW), jnp.float32)

    sums, cnts = _sc_partial_sums(features, index2d, zero_sums, zero_cnts,
                                  ones_blk)

    out = pl.pallas_call(
        _combine_body,
        out_shape=jax.ShapeDtypeStruct((V, D), jnp.float32),
    )(sums, cnts)
    return out


# speedup vs baseline: 2.3781x; 2.3781x over previous
import functools

import jax
import jax.numpy as jnp
from jax import lax
from jax.experimental import pallas as pl
from jax.experimental.pallas import tpu as pltpu
from jax.experimental.pallas import tpu_sc as plsc

N_POINTS = 320000
D = 128
V = 10000
NC = 2
NS = 16
HALF = V // NC
VPL = 5120
ROWS_PER_TILE = N_POINTS // NS
CHUNK = 80
CHUNKS_PER_TILE = ROWS_PER_TILE // CHUNK
V_PER_TILE = VPL // NS
V_STEPS = V_PER_TILE // CHUNK
CW = 128
L = 16


def _sc_partial_sums(features, index, zero_rows, zero_cnts, ones_blk):
    mesh = plsc.VectorSubcoreMesh(core_axis_name="c", subcore_axis_name="s")

    @functools.partial(
        pl.kernel,
        out_type=(
            jax.ShapeDtypeStruct((NC * VPL, D), jnp.float32),
            jax.ShapeDtypeStruct((NC * VPL, CW), jnp.float32),
        ),
        mesh=mesh,
        scratch_types=(
            pltpu.VMEM((CHUNK, D), jnp.float32),
            pltpu.VMEM((CHUNK,), jnp.int32),
            pltpu.VMEM((CHUNK,), jnp.int32),
            pltpu.VMEM((CHUNK, CW), jnp.float32),
            pltpu.VMEM((CHUNK, CW), jnp.float32),
            pltpu.VMEM_SHARED((VPL, D), jnp.float32),
            pltpu.VMEM_SHARED((VPL, CW), jnp.float32),
        ),
    )
    def body(feat_hbm, idx_hbm, zr_hbm, zc_hbm, ones_hbm, sums_out, cnts_out,
             rows_v, idx_v, idx2_v, ones_v, cnt_v, acc_s, cacc_s):
        c = lax.axis_index("c")
        s = lax.axis_index("s")
        v0 = s * V_PER_TILE
        half_base = c * HALF

        pltpu.sync_copy(zr_hbm, rows_v)
        pltpu.sync_copy(zc_hbm, cnt_v)
        for j in range(V_STEPS):
            pltpu.sync_copy(rows_v, acc_s.at[pl.ds(v0 + j * CHUNK, CHUNK)])
            pltpu.sync_copy(cnt_v, cacc_s.at[pl.ds(v0 + j * CHUNK, CHUNK)])
        pltpu.sync_copy(ones_hbm, ones_v)
        plsc.subcore_barrier()

        def chunk_body(k, carry):
            row0 = s * ROWS_PER_TILE + k * CHUNK
            pltpu.sync_copy(feat_hbm.at[pl.ds(row0, CHUNK)], rows_v)
            pltpu.sync_copy(idx_hbm.at[pl.ds(row0, CHUNK)], idx_v)
            for j in range(CHUNK // L):
                raw = idx_v[pl.ds(j * L, L)]
                local = raw - half_base
                ok = (local >= 0) & (local < HALF)
                idx2_v[pl.ds(j * L, L)] = jnp.where(ok, local, HALF)
            pltpu.sync_copy(rows_v, acc_s.at[idx2_v], add=True)
            pltpu.sync_copy(ones_v, cacc_s.at[idx2_v], add=True)
            return carry

        lax.fori_loop(0, CHUNKS_PER_TILE, chunk_body, 0)
        plsc.subcore_barrier()

        for j in range(V_STEPS):
            src0 = v0 + j * CHUNK
            dst0 = c * VPL + v0 + j * CHUNK
            pltpu.sync_copy(acc_s.at[pl.ds(src0, CHUNK)], rows_v)
            pltpu.sync_copy(rows_v, sums_out.at[pl.ds(dst0, CHUNK)])
            pltpu.sync_copy(cacc_s.at[pl.ds(src0, CHUNK)], cnt_v)
            pltpu.sync_copy(cnt_v, cnts_out.at[pl.ds(dst0, CHUNK)])

    return body(features, index, zero_rows, zero_cnts, ones_blk)


def _combine_body(sums_ref, cnts_ref, out_ref):
    total = jnp.concatenate(
        [sums_ref[0:HALF], sums_ref[VPL:VPL + HALF]], axis=0)
    cnt = jnp.concatenate(
        [cnts_ref[0:HALF, 0], cnts_ref[VPL:VPL + HALF, 0]], axis=0)
    out_ref[...] = total / jnp.clip(cnt, 1.0, None)[:, None]


def kernel(features, index):
    index = index.astype(jnp.int32)
    zero_rows = jnp.zeros((CHUNK, D), jnp.float32)
    zero_cnts = jnp.zeros((CHUNK, CW), jnp.float32)
    ones_blk = jnp.ones((CHUNK, CW), jnp.float32)

    sums, cnts = _sc_partial_sums(features, index, zero_rows, zero_cnts,
                                  ones_blk)

    out = pl.pallas_call(
        _combine_body,
        out_shape=jax.ShapeDtypeStruct((V, D), jnp.float32),
    )(sums, cnts)
    return out


# speedup vs baseline: 3.0741x; 1.2927x over previous
import functools

import jax
import jax.numpy as jnp
from jax import lax
from jax.experimental import pallas as pl
from jax.experimental.pallas import tpu as pltpu
from jax.experimental.pallas import tpu_sc as plsc

N_POINTS = 320000
D = 128
V = 10000
NC = 2
NS = 16
HALF = V // NC
VPL = 5120
ROWS_PER_TILE = N_POINTS // NS
CHUNK = 80
CHUNKS_PER_TILE = ROWS_PER_TILE // CHUNK
V_PER_TILE = VPL // NS
V_STEPS = V_PER_TILE // CHUNK
CW = 128
L = 16


def _sc_partial_sums(features, index, zero_rows, zero_cnts, ones_blk):
    mesh = plsc.VectorSubcoreMesh(core_axis_name="c", subcore_axis_name="s")

    @functools.partial(
        pl.kernel,
        out_type=(
            jax.ShapeDtypeStruct((NC * VPL, D), jnp.float32),
            jax.ShapeDtypeStruct((NC * VPL, CW), jnp.float32),
        ),
        mesh=mesh,
        scratch_types=(
            pltpu.VMEM((CHUNK, D), jnp.float32),
            pltpu.VMEM((CHUNK, D), jnp.float32),
            pltpu.VMEM((CHUNK,), jnp.int32),
            pltpu.VMEM((CHUNK,), jnp.int32),
            pltpu.VMEM((CHUNK,), jnp.int32),
            pltpu.VMEM((CHUNK,), jnp.int32),
            pltpu.VMEM((CHUNK, CW), jnp.float32),
            pltpu.VMEM((CHUNK, CW), jnp.float32),
            pltpu.VMEM_SHARED((VPL, D), jnp.float32),
            pltpu.VMEM_SHARED((VPL, CW), jnp.float32),
            pltpu.SemaphoreType.DMA,
            pltpu.SemaphoreType.DMA,
        ),
    )
    def body(feat_hbm, idx_hbm, zr_hbm, zc_hbm, ones_hbm, sums_out, cnts_out,
             rows_a, rows_b, idx_a, idx_b, idx2_a, idx2_b, ones_v, cnt_v,
             acc_s, cacc_s, sem_a, sem_b):
        c = lax.axis_index("c")
        s = lax.axis_index("s")
        v0 = s * V_PER_TILE
        half_base = c * HALF

        pltpu.sync_copy(zr_hbm, rows_a)
        pltpu.sync_copy(zc_hbm, cnt_v)
        for j in range(V_STEPS):
            pltpu.sync_copy(rows_a, acc_s.at[pl.ds(v0 + j * CHUNK, CHUNK)])
            pltpu.sync_copy(cnt_v, cacc_s.at[pl.ds(v0 + j * CHUNK, CHUNK)])
        pltpu.sync_copy(ones_hbm, ones_v)
        plsc.subcore_barrier()

        def chunk_slice(k):
            row0 = lax.min(s * ROWS_PER_TILE + k * CHUNK, N_POINTS - CHUNK)
            return pl.ds(row0, CHUNK)

        def fetch(k, rows_x, idx_x, sem):
            sl = chunk_slice(k)
            pltpu.async_copy(feat_hbm.at[sl], rows_x, sem)
            pltpu.async_copy(idx_hbm.at[sl], idx_x, sem)

        def wait_fetch(k, rows_x, idx_x, sem):
            sl = chunk_slice(k)
            pltpu.make_async_copy(feat_hbm.at[sl], rows_x, sem).wait()
            pltpu.make_async_copy(idx_hbm.at[sl], idx_x, sem).wait()

        def process(rows_x, idx_x, idx2_x):
            for j in range(CHUNK // L):
                raw = idx_x[pl.ds(j * L, L)]
                local = raw - half_base
                ok = (local >= 0) & (local < HALF)
                idx2_x[pl.ds(j * L, L)] = jnp.where(ok, local, HALF)
            pltpu.sync_copy(rows_x, acc_s.at[idx2_x], add=True)
            pltpu.sync_copy(ones_v, cacc_s.at[idx2_x], add=True)

        fetch(0, rows_a, idx_a, sem_a)

        def chunk_body(k, carry):
            k2 = 2 * k
            wait_fetch(k2, rows_a, idx_a, sem_a)
            fetch(k2 + 1, rows_b, idx_b, sem_b)
            process(rows_a, idx_a, idx2_a)
            wait_fetch(k2 + 1, rows_b, idx_b, sem_b)
            fetch(k2 + 2, rows_a, idx_a, sem_a)
            process(rows_b, idx_b, idx2_b)
            return carry

        lax.fori_loop(0, CHUNKS_PER_TILE // 2, chunk_body, 0)
        wait_fetch(CHUNKS_PER_TILE, rows_a, idx_a, sem_a)
        plsc.subcore_barrier()

        for j in range(V_STEPS):
            src0 = v0 + j * CHUNK
            dst0 = c * VPL + v0 + j * CHUNK
            pltpu.sync_copy(acc_s.at[pl.ds(src0, CHUNK)], rows_a)
            pltpu.sync_copy(rows_a, sums_out.at[pl.ds(dst0, CHUNK)])
            pltpu.sync_copy(cacc_s.at[pl.ds(src0, CHUNK)], cnt_v)
            pltpu.sync_copy(cnt_v, cnts_out.at[pl.ds(dst0, CHUNK)])

    return body(features, index, zero_rows, zero_cnts, ones_blk)


def _combine_body(sums_ref, cnts_ref, out_ref):
    total = jnp.concatenate(
        [sums_ref[0:HALF], sums_ref[VPL:VPL + HALF]], axis=0)
    cnt = jnp.concatenate(
        [cnts_ref[0:HALF, 0], cnts_ref[VPL:VPL + HALF, 0]], axis=0)
    out_ref[...] = total / jnp.clip(cnt, 1.0, None)[:, None]


def kernel(features, index):
    index = index.astype(jnp.int32)
    zero_rows = jnp.zeros((CHUNK, D), jnp.float32)
    zero_cnts = jnp.zeros((CHUNK, CW), jnp.float32)
    ones_blk = jnp.ones((CHUNK, CW), jnp.float32)

    sums, cnts = _sc_partial_sums(features, index, zero_rows, zero_cnts,
                                  ones_blk)

    out = pl.pallas_call(
        _combine_body,
        out_shape=jax.ShapeDtypeStruct((V, D), jnp.float32),
    )(sums, cnts)
    return out


# speedup vs baseline: 3.0746x; 1.0002x over previous
import functools

import jax
import jax.numpy as jnp
from jax import lax
from jax.experimental import pallas as pl
from jax.experimental.pallas import tpu as pltpu
from jax.experimental.pallas import tpu_sc as plsc

N_POINTS = 320000
D = 128
V = 10000
NC = 2
NS = 16
HALF = V // NC
VPL = 5120
ROWS_PER_TILE = N_POINTS // NS
CHUNK = 80
CHUNKS_PER_TILE = ROWS_PER_TILE // CHUNK
V_PER_TILE = VPL // NS
V_STEPS = V_PER_TILE // CHUNK
CW = 128
L = 16


def _sc_partial_sums(features, index, zero_rows, zero_cnts, ones_blk):
    mesh = plsc.VectorSubcoreMesh(core_axis_name="c", subcore_axis_name="s")

    @functools.partial(
        pl.kernel,
        out_type=(
            jax.ShapeDtypeStruct((NC * VPL, D), jnp.float32),
            jax.ShapeDtypeStruct((NC * VPL, CW), jnp.float32),
        ),
        mesh=mesh,
        scratch_types=(
            pltpu.VMEM((CHUNK, D), jnp.float32),
            pltpu.VMEM((CHUNK, D), jnp.float32),
            pltpu.VMEM((CHUNK,), jnp.int32),
            pltpu.VMEM((CHUNK,), jnp.int32),
            pltpu.VMEM((CHUNK,), jnp.int32),
            pltpu.VMEM((CHUNK,), jnp.int32),
            pltpu.VMEM((CHUNK, CW), jnp.float32),
            pltpu.VMEM((CHUNK, CW), jnp.float32),
            pltpu.VMEM_SHARED((VPL, D), jnp.float32),
            pltpu.VMEM_SHARED((VPL, CW), jnp.float32),
            pltpu.SemaphoreType.DMA,
            pltpu.SemaphoreType.DMA,
            pltpu.SemaphoreType.DMA,
        ),
    )
    def body(feat_hbm, idx_hbm, zr_hbm, zc_hbm, ones_hbm, sums_out, cnts_out,
             rows_a, rows_b, idx_a, idx_b, idx2_a, idx2_b, ones_v, cnt_v,
             acc_s, cacc_s, sem_a, sem_b, sem_s):
        c = lax.axis_index("c")
        s = lax.axis_index("s")
        v0 = s * V_PER_TILE
        half_base = c * HALF

        pltpu.sync_copy(zr_hbm, rows_a)
        pltpu.sync_copy(zc_hbm, cnt_v)
        for j in range(V_STEPS):
            pltpu.sync_copy(rows_a, acc_s.at[pl.ds(v0 + j * CHUNK, CHUNK)])
            pltpu.sync_copy(cnt_v, cacc_s.at[pl.ds(v0 + j * CHUNK, CHUNK)])
        pltpu.sync_copy(ones_hbm, ones_v)
        plsc.subcore_barrier()

        def chunk_slice(k):
            row0 = lax.min(s * ROWS_PER_TILE + k * CHUNK, N_POINTS - CHUNK)
            return pl.ds(row0, CHUNK)

        def fetch(k, rows_x, idx_x, sem):
            sl = chunk_slice(k)
            pltpu.async_copy(feat_hbm.at[sl], rows_x, sem)
            pltpu.async_copy(idx_hbm.at[sl], idx_x, sem)

        def wait_fetch(k, rows_x, idx_x, sem):
            sl = chunk_slice(k)
            pltpu.make_async_copy(feat_hbm.at[sl], rows_x, sem).wait()
            pltpu.make_async_copy(idx_hbm.at[sl], idx_x, sem).wait()

        def process(rows_x, idx_x, idx2_x):
            for j in range(CHUNK // L):
                raw = idx_x[pl.ds(j * L, L)]
                local = raw - half_base
                ok = (local >= 0) & (local < HALF)
                idx2_x[pl.ds(j * L, L)] = jnp.where(ok, local, HALF)
            a = pltpu.async_copy(rows_x, acc_s.at[idx2_x], sem_s, add=True)
            b = pltpu.async_copy(ones_v, cacc_s.at[idx2_x], sem_s, add=True)
            a.wait()
            b.wait()

        fetch(0, rows_a, idx_a, sem_a)

        def chunk_body(k, carry):
            k2 = 2 * k
            wait_fetch(k2, rows_a, idx_a, sem_a)
            fetch(k2 + 1, rows_b, idx_b, sem_b)
            process(rows_a, idx_a, idx2_a)
            wait_fetch(k2 + 1, rows_b, idx_b, sem_b)
            fetch(k2 + 2, rows_a, idx_a, sem_a)
            process(rows_b, idx_b, idx2_b)
            return carry

        lax.fori_loop(0, CHUNKS_PER_TILE // 2, chunk_body, 0)
        wait_fetch(CHUNKS_PER_TILE, rows_a, idx_a, sem_a)
        plsc.subcore_barrier()

        for j in range(V_STEPS):
            src0 = v0 + j * CHUNK
            dst0 = c * VPL + v0 + j * CHUNK
            pltpu.sync_copy(acc_s.at[pl.ds(src0, CHUNK)], rows_a)
            pltpu.sync_copy(rows_a, sums_out.at[pl.ds(dst0, CHUNK)])
            pltpu.sync_copy(cacc_s.at[pl.ds(src0, CHUNK)], cnt_v)
            pltpu.sync_copy(cnt_v, cnts_out.at[pl.ds(dst0, CHUNK)])

    return body(features, index, zero_rows, zero_cnts, ones_blk)


def _combine_body(sums_ref, cnts_ref, out_ref):
    total = jnp.concatenate(
        [sums_ref[0:HALF], sums_ref[VPL:VPL + HALF]], axis=0)
    cnt = jnp.concatenate(
        [cnts_ref[0:HALF, 0], cnts_ref[VPL:VPL + HALF, 0]], axis=0)
    out_ref[...] = total / jnp.clip(cnt, 1.0, None)[:, None]


def kernel(features, index):
    index = index.astype(jnp.int32)
    zero_rows = jnp.zeros((CHUNK, D), jnp.float32)
    zero_cnts = jnp.zeros((CHUNK, CW), jnp.float32)
    ones_blk = jnp.ones((CHUNK, CW), jnp.float32)

    sums, cnts = _sc_partial_sums(features, index, zero_rows, zero_cnts,
                                  ones_blk)

    out = pl.pallas_call(
        _combine_body,
        out_shape=jax.ShapeDtypeStruct((V, D), jnp.float32),
    )(sums, cnts)
    return out
